# baseline (speedup 1.0000x reference)
import jax
import jax.numpy as jnp
from jax.experimental import pallas as pl
from jax.experimental.pallas import tpu as pltpu

SEQ_BLOCK = 512


def _add_body(x_ref, pe_ref, o_ref):
    o_ref[...] = x_ref[...] + pe_ref[...][None, :, :]


def kernel(x, pe_weight):
    batch, seq_len, d_model = x.shape
    pe = pe_weight[:seq_len]
    num_seq_blocks = seq_len // SEQ_BLOCK

    return pl.pallas_call(
        _add_body,
        grid=(num_seq_blocks,),
        in_specs=[
            pl.BlockSpec((batch, SEQ_BLOCK, d_model), lambda i: (0, i, 0)),
            pl.BlockSpec((SEQ_BLOCK, d_model), lambda i: (i, 0)),
        ],
        out_specs=pl.BlockSpec((batch, SEQ_BLOCK, d_model), lambda i: (0, i, 0)),
        out_shape=jax.ShapeDtypeStruct(x.shape, x.dtype),
        compiler_params=pltpu.CompilerParams(
            dimension_semantics=("arbitrary",),
            vmem_limit_bytes=128 * 1024 * 1024,
        ),
    )(x, pe)


# speedup vs baseline: 1.0031x; 1.0031x over previous
import jax
import jax.numpy as jnp
from jax.experimental import pallas as pl
from jax.experimental.pallas import tpu as pltpu

CHUNK = 512
NBUF = 4


def kernel(x, pe_weight):
    batch, seq_len, d = x.shape
    rows = batch * seq_len
    num_sblk = seq_len // CHUNK
    total = num_sblk * batch
    assert seq_len % CHUNK == 0

    x2 = x.reshape(rows, d)
    pe2 = pe_weight[:seq_len]

    def body(x_hbm, pe_hbm, o_hbm, xb, ob, peb, xs, os_, ps):
        def x_copy(t, slot):
            sblk, b = divmod(t, batch)
            row0 = b * seq_len + sblk * CHUNK
            return pltpu.make_async_copy(
                x_hbm.at[pl.ds(row0, CHUNK)], xb.at[slot], xs.at[slot])

        def pe_copy(sblk, slot):
            return pltpu.make_async_copy(
                pe_hbm.at[pl.ds(sblk * CHUNK, CHUNK)], peb.at[slot], ps.at[slot])

        def o_copy(t, slot):
            sblk, b = divmod(t, batch)
            row0 = b * seq_len + sblk * CHUNK
            return pltpu.make_async_copy(
                ob.at[slot], o_hbm.at[pl.ds(row0, CHUNK)], os_.at[slot])

        pe_copy(0, 0).start()
        if num_sblk > 1:
            pe_copy(1, 1).start()
        for t in range(min(NBUF, total)):
            x_copy(t, t % NBUF).start()

        for t in range(total):
            sblk, b = divmod(t, batch)
            slot = t % NBUF
            if b == 0:
                pe_copy(sblk, sblk % 2).wait()
            x_copy(t, slot).wait()
            if t >= NBUF:
                o_copy(t - NBUF, slot).wait()
            ob[slot] = xb[slot] + peb[sblk % 2]
            o_copy(t, slot).start()
            if t + NBUF < total:
                x_copy(t + NBUF, slot).start()
            if b == batch - 1 and sblk + 2 < num_sblk:
                pe_copy(sblk + 2, sblk % 2).start()

        for t in range(max(0, total - NBUF), total):
            o_copy(t, t % NBUF).wait()

    out = pl.pallas_call(
        body,
        in_specs=[
            pl.BlockSpec(memory_space=pl.ANY),
            pl.BlockSpec(memory_space=pl.ANY),
        ],
        out_specs=pl.BlockSpec(memory_space=pl.ANY),
        out_shape=jax.ShapeDtypeStruct((rows, d), x.dtype),
        scratch_shapes=[
            pltpu.VMEM((NBUF, CHUNK, d), jnp.float32),
            pltpu.VMEM((NBUF, CHUNK, d), jnp.float32),
            pltpu.VMEM((2, CHUNK, d), jnp.float32),
            pltpu.SemaphoreType.DMA((NBUF,)),
            pltpu.SemaphoreType.DMA((NBUF,)),
            pltpu.SemaphoreType.DMA((2,)),
        ],
        compiler_params=pltpu.CompilerParams(
            vmem_limit_bytes=64 * 1024 * 1024,
        ),
    )(x2, pe2)
    return out.reshape(batch, seq_len, d)


# speedup vs baseline: 1.0090x; 1.0059x over previous
import jax
import jax.numpy as jnp
from jax.experimental import pallas as pl
from jax.experimental.pallas import tpu as pltpu

SEQ_BLOCK = 2048


def _add_body(x_ref, pe_ref, o_ref):
    o_ref[...] = x_ref[...] + pe_ref[...][None, :, :]


def kernel(x, pe_weight):
    batch, seq_len, d_model = x.shape
    pe = pe_weight[:seq_len]
    num_seq_blocks = seq_len // SEQ_BLOCK

    grid = (num_seq_blocks, batch)
    return pl.pallas_call(
        _add_body,
        grid=grid,
        in_specs=[
            pl.BlockSpec((1, SEQ_BLOCK, d_model), lambda i, j: (j, i, 0)),
            pl.BlockSpec((SEQ_BLOCK, d_model), lambda i, j: (i, 0)),
        ],
        out_specs=pl.BlockSpec((1, SEQ_BLOCK, d_model), lambda i, j: (j, i, 0)),
        out_shape=jax.ShapeDtypeStruct(x.shape, x.dtype),
        compiler_params=pltpu.CompilerParams(
            dimension_semantics=("arbitrary", "arbitrary"),
        ),
    )(x, pe)
